# baseline (device time: 708385 ns/iter reference)
import jax
import jax.numpy as jnp
from jax import lax
from jax.experimental import pallas as pl
from jax.experimental.pallas import tpu as pltpu

N_DEV = 4
M_BLK = 2048


def _a2a(x):
    k_tot, k_loc = x.shape

    def body(x_ref, xg_ref, send_sems, recv_sems, copy_sem):
        my = lax.axis_index("i")

        barrier = pltpu.get_barrier_semaphore()
        for d in (1, 2, 3):
            t = lax.rem(my + d, N_DEV)
            pl.semaphore_signal(
                barrier, inc=1, device_id=(t,),
                device_id_type=pl.DeviceIdType.MESH,
            )
        pl.semaphore_wait(barrier, N_DEV - 1)

        local = pltpu.make_async_copy(
            x_ref.at[pl.ds(my * M_BLK, M_BLK), :], xg_ref.at[my], copy_sem
        )
        local.start()

        sends = []
        for d in (1, 2, 3):
            t = lax.rem(my + d, N_DEV)
            rdma = pltpu.make_async_remote_copy(
                src_ref=x_ref.at[pl.ds(t * M_BLK, M_BLK), :],
                dst_ref=xg_ref.at[my],
                send_sem=send_sems.at[d - 1],
                recv_sem=recv_sems.at[d - 1],
                device_id=(t,),
                device_id_type=pl.DeviceIdType.MESH,
            )
            rdma.start()
            sends.append(rdma)

        local.wait()
        for rdma in sends:
            rdma.wait_send()

        for d in (1, 2, 3):
            s = lax.rem(my - d + N_DEV, N_DEV)
            recv = pltpu.make_async_remote_copy(
                src_ref=x_ref.at[pl.ds(s * M_BLK, M_BLK), :],
                dst_ref=xg_ref.at[s],
                send_sem=send_sems.at[d - 1],
                recv_sem=recv_sems.at[d - 1],
                device_id=(s,),
                device_id_type=pl.DeviceIdType.MESH,
            )
            recv.wait_recv()

    return pl.pallas_call(
        body,
        out_shape=jax.ShapeDtypeStruct((N_DEV, M_BLK, k_loc), x.dtype),
        in_specs=[pl.BlockSpec(memory_space=pl.ANY)],
        out_specs=pl.BlockSpec(memory_space=pl.ANY),
        scratch_shapes=[
            pltpu.SemaphoreType.DMA((N_DEV - 1,)),
            pltpu.SemaphoreType.DMA((N_DEV - 1,)),
            pltpu.SemaphoreType.DMA,
        ],
        compiler_params=pltpu.CompilerParams(collective_id=0),
    )(x)


def kernel(x, w_mat):
    xg = _a2a(x)
    w_r = w_mat.reshape(N_DEV, M_BLK, w_mat.shape[1])
    y = jnp.einsum(
        "jmk,jkn->mn", xg, w_r, preferred_element_type=jnp.float32
    )
    return y * jax.nn.sigmoid(y)


# device time: 501445 ns/iter; 1.4127x vs baseline; 1.4127x over previous
import jax
import jax.numpy as jnp
from jax import lax
from jax.experimental import pallas as pl
from jax.experimental.pallas import tpu as pltpu

jax.config.update("jax_compilation_cache_dir", "/tmp/jax_cache")
jax.config.update("jax_persistent_cache_min_compile_time_secs", 0.0)

N_DEV = 4
M_BLK = 2048
NT = 512
N_TILES = 8

_SRC_OFF = (0, 3, 1, 2)
_SEM_FOR = (None, 0, 2, 1)


def kernel(x, w_mat):
    k_tot, k_loc = x.shape
    n_out = w_mat.shape[1]
    assert k_loc == M_BLK and n_out == N_TILES * NT

    def body(x_ref, w_ref, out_ref, xg_ref, x_buf, w_buf, acc_ref,
             send_sems, recv_sems, x_sem, w_sems, out_sem):
        my = lax.axis_index("i")

        def src_of(j):
            return lax.rem(my + _SRC_OFF[j], N_DEV)

        barrier = pltpu.get_barrier_semaphore()
        for d in (1, 2, 3):
            t = lax.rem(my + d, N_DEV)
            pl.semaphore_signal(
                barrier, inc=1, device_id=(t,),
                device_id_type=pl.DeviceIdType.MESH,
            )
        pl.semaphore_wait(barrier, N_DEV - 1)

        sends = []
        for d in (1, 2, 3):
            t = lax.rem(my + d, N_DEV)
            rdma = pltpu.make_async_remote_copy(
                src_ref=x_ref.at[pl.ds(t * M_BLK, M_BLK), :],
                dst_ref=xg_ref.at[my],
                send_sem=send_sems.at[d - 1],
                recv_sem=recv_sems.at[d - 1],
                device_id=(t,),
                device_id_type=pl.DeviceIdType.MESH,
            )
            rdma.start()
            sends.append(rdma)

        def wait_recv(j):
            s = src_of(j)
            sem = _SEM_FOR[j]
            pltpu.make_async_remote_copy(
                src_ref=x_ref.at[pl.ds(s * M_BLK, M_BLK), :],
                dst_ref=xg_ref.at[s],
                send_sem=send_sems.at[sem],
                recv_sem=recv_sems.at[sem],
                device_id=(s,),
                device_id_type=pl.DeviceIdType.MESH,
            ).wait_recv()

        def xdma(j):
            if j == 0:
                src = x_ref.at[pl.ds(my * M_BLK, M_BLK), :]
            else:
                src = xg_ref.at[src_of(j)]
            return pltpu.make_async_copy(src, x_buf, x_sem)

        def wdma_start(s, n, slot):
            pltpu.make_async_copy(
                w_ref.at[pl.ds(s * M_BLK, M_BLK), pl.ds(n * NT, NT)],
                w_buf.at[slot], w_sems.at[slot],
            ).start()

        def wdma_wait(slot):
            pltpu.make_async_copy(
                w_ref.at[pl.ds(0, M_BLK), pl.ds(0, NT)],
                w_buf.at[slot], w_sems.at[slot],
            ).wait()

        xdma(0).start()
        s0 = src_of(0)
        wdma_start(s0, 0, 0)
        wdma_start(s0, 1, 1)

        for j in range(N_DEV):
            s = src_of(j)
            xdma(j).wait()

            def n_body(n, _, j=j, s=s):
                slot = lax.rem(n, 2)
                wdma_wait(slot)
                prod = jnp.dot(
                    x_buf[:, :], w_buf[slot],
                    preferred_element_type=jnp.float32,
                )
                if j == 0:
                    acc_ref[:, pl.ds(n * NT, NT)] = prod
                else:
                    acc_ref[:, pl.ds(n * NT, NT)] += prod

                @pl.when(n + 2 < N_TILES)
                def _():
                    wdma_start(s, n + 2, slot)

                return 0

            lax.fori_loop(0, N_TILES, n_body, 0)

            if j + 1 < N_DEV:
                s_next = src_of(j + 1)
                wdma_start(s_next, 0, 0)
                wdma_start(s_next, 1, 1)
                wait_recv(j + 1)
                xdma(j + 1).start()

        def silu_body(n, _):
            v = acc_ref[:, pl.ds(n * NT, NT)]
            acc_ref[:, pl.ds(n * NT, NT)] = v * (1.0 / (1.0 + jnp.exp(-v)))
            return 0

        lax.fori_loop(0, N_TILES, silu_body, 0)

        out_cp = pltpu.make_async_copy(acc_ref, out_ref, out_sem)
        out_cp.start()
        out_cp.wait()

        for rdma in sends:
            rdma.wait_send()

    y, _ = pl.pallas_call(
        body,
        out_shape=[
            jax.ShapeDtypeStruct((M_BLK, n_out), jnp.float32),
            jax.ShapeDtypeStruct((N_DEV, M_BLK, k_loc), jnp.float32),
        ],
        in_specs=[
            pl.BlockSpec(memory_space=pl.ANY),
            pl.BlockSpec(memory_space=pl.ANY),
        ],
        out_specs=[
            pl.BlockSpec(memory_space=pl.ANY),
            pl.BlockSpec(memory_space=pl.ANY),
        ],
        scratch_shapes=[
            pltpu.VMEM((M_BLK, k_loc), jnp.float32),
            pltpu.VMEM((2, M_BLK, NT), jnp.float32),
            pltpu.VMEM((M_BLK, n_out), jnp.float32),
            pltpu.SemaphoreType.DMA((N_DEV - 1,)),
            pltpu.SemaphoreType.DMA((N_DEV - 1,)),
            pltpu.SemaphoreType.DMA,
            pltpu.SemaphoreType.DMA((2,)),
            pltpu.SemaphoreType.DMA,
        ],
        compiler_params=pltpu.CompilerParams(
            collective_id=0,
            vmem_limit_bytes=63 * 1024 * 1024,
        ),
    )(x, w_mat)
    return y


# device time: 350005 ns/iter; 2.0239x vs baseline; 1.4327x over previous
import jax
import jax.numpy as jnp
from jax import lax
from jax.experimental import pallas as pl
from jax.experimental.pallas import tpu as pltpu

jax.config.update("jax_compilation_cache_dir", "/tmp/jax_cache")
jax.config.update("jax_persistent_cache_min_compile_time_secs", 0.0)

N_DEV = 4
M_BLK = 2048
NT = 512
N_TILES = 8

_SRC_OFF = (0, 3, 1, 2)
_SEM_FOR = (None, 0, 2, 1)


def kernel(x, w_mat):
    k_tot, k_loc = x.shape
    n_out = w_mat.shape[1]
    assert k_loc == M_BLK and n_out == N_TILES * NT

    x = x.astype(jnp.bfloat16)

    def body(x_ref, w_ref, out_ref, xg_ref, x_buf, w_buf, acc_ref,
             send_sems, recv_sems, x_sem, w_sems, out_sem):
        my = lax.axis_index("i")

        def src_of(j):
            return lax.rem(my + _SRC_OFF[j], N_DEV)

        barrier = pltpu.get_barrier_semaphore()
        for d in (1, 2, 3):
            t = lax.rem(my + d, N_DEV)
            pl.semaphore_signal(
                barrier, inc=1, device_id=(t,),
                device_id_type=pl.DeviceIdType.MESH,
            )
        pl.semaphore_wait(barrier, N_DEV - 1)

        sends = []
        for d in (1, 2, 3):
            t = lax.rem(my + d, N_DEV)
            rdma = pltpu.make_async_remote_copy(
                src_ref=x_ref.at[pl.ds(t * M_BLK, M_BLK), :],
                dst_ref=xg_ref.at[my],
                send_sem=send_sems.at[d - 1],
                recv_sem=recv_sems.at[d - 1],
                device_id=(t,),
                device_id_type=pl.DeviceIdType.MESH,
            )
            rdma.start()
            sends.append(rdma)

        def wait_recv(j):
            s = src_of(j)
            sem = _SEM_FOR[j]
            pltpu.make_async_remote_copy(
                src_ref=x_ref.at[pl.ds(s * M_BLK, M_BLK), :],
                dst_ref=xg_ref.at[s],
                send_sem=send_sems.at[sem],
                recv_sem=recv_sems.at[sem],
                device_id=(s,),
                device_id_type=pl.DeviceIdType.MESH,
            ).wait_recv()

        def xdma(j):
            if j == 0:
                src = x_ref.at[pl.ds(my * M_BLK, M_BLK), :]
            else:
                src = xg_ref.at[src_of(j)]
            return pltpu.make_async_copy(src, x_buf, x_sem)

        def wdma_start(s, n, slot):
            pltpu.make_async_copy(
                w_ref.at[pl.ds(s * M_BLK, M_BLK), pl.ds(n * NT, NT)],
                w_buf.at[slot], w_sems.at[slot],
            ).start()

        def wdma_wait(slot):
            pltpu.make_async_copy(
                w_ref.at[pl.ds(0, M_BLK), pl.ds(0, NT)],
                w_buf.at[slot], w_sems.at[slot],
            ).wait()

        xdma(0).start()
        s0 = src_of(0)
        wdma_start(s0, 0, 0)
        wdma_start(s0, 1, 1)

        for j in range(N_DEV):
            s = src_of(j)
            xdma(j).wait()

            def n_body(n, _, j=j, s=s):
                slot = lax.rem(n, 2)
                wdma_wait(slot)
                prod = jnp.dot(
                    x_buf[:, :], w_buf[slot].astype(jnp.bfloat16),
                    preferred_element_type=jnp.float32,
                )
                if j == 0:
                    acc_ref[:, pl.ds(n * NT, NT)] = prod
                else:
                    acc_ref[:, pl.ds(n * NT, NT)] += prod

                @pl.when(n + 2 < N_TILES)
                def _():
                    wdma_start(s, n + 2, slot)

                return 0

            lax.fori_loop(0, N_TILES, n_body, 0)

            if j + 1 < N_DEV:
                s_next = src_of(j + 1)
                wdma_start(s_next, 0, 0)
                wdma_start(s_next, 1, 1)
                wait_recv(j + 1)
                xdma(j + 1).start()

        def silu_body(n, _):
            v = acc_ref[:, pl.ds(n * NT, NT)]
            acc_ref[:, pl.ds(n * NT, NT)] = v * (1.0 / (1.0 + jnp.exp(-v)))
            return 0

        lax.fori_loop(0, N_TILES, silu_body, 0)

        out_cp = pltpu.make_async_copy(acc_ref, out_ref, out_sem)
        out_cp.start()
        out_cp.wait()

        for rdma in sends:
            rdma.wait_send()

    y, _ = pl.pallas_call(
        body,
        out_shape=[
            jax.ShapeDtypeStruct((M_BLK, n_out), jnp.float32),
            jax.ShapeDtypeStruct((N_DEV, M_BLK, k_loc), jnp.bfloat16),
        ],
        in_specs=[
            pl.BlockSpec(memory_space=pl.ANY),
            pl.BlockSpec(memory_space=pl.ANY),
        ],
        out_specs=[
            pl.BlockSpec(memory_space=pl.ANY),
            pl.BlockSpec(memory_space=pl.ANY),
        ],
        scratch_shapes=[
            pltpu.VMEM((M_BLK, k_loc), jnp.bfloat16),
            pltpu.VMEM((2, M_BLK, NT), jnp.float32),
            pltpu.VMEM((M_BLK, n_out), jnp.float32),
            pltpu.SemaphoreType.DMA((N_DEV - 1,)),
            pltpu.SemaphoreType.DMA((N_DEV - 1,)),
            pltpu.SemaphoreType.DMA,
            pltpu.SemaphoreType.DMA((2,)),
            pltpu.SemaphoreType.DMA,
        ],
        compiler_params=pltpu.CompilerParams(
            collective_id=0,
            vmem_limit_bytes=63 * 1024 * 1024,
        ),
    )(x, w_mat)
    return y


# device time: 339366 ns/iter; 2.0874x vs baseline; 1.0313x over previous
import jax
import jax.numpy as jnp
from jax import lax
from jax.experimental import pallas as pl
from jax.experimental.pallas import tpu as pltpu

jax.config.update("jax_compilation_cache_dir", "/tmp/jax_cache")
jax.config.update("jax_persistent_cache_min_compile_time_secs", 0.0)

N_DEV = 4
M_BLK = 2048
MH = 1024
NT = 512
N_TILES = 8

_SRC_OFF = (0, 3, 1, 2)
_SEM_FOR = (None, 0, 2, 1)


def kernel(x, w_mat):
    k_tot, k_loc = x.shape
    n_out = w_mat.shape[1]
    assert k_loc == M_BLK and n_out == N_TILES * NT

    x = x.astype(jnp.bfloat16)

    def body(x_ref, w_ref, out_ref, xg_ref, x_full, x_half, w_buf, acc_ref,
             send_sems, recv_sems, x_sem, w_sems, out_sems):
        my = lax.axis_index("i")

        def src_of(j):
            return lax.rem(my + _SRC_OFF[j], N_DEV)

        barrier = pltpu.get_barrier_semaphore()
        for d in (1, 2, 3):
            t = lax.rem(my + d, N_DEV)
            pl.semaphore_signal(
                barrier, inc=1, device_id=(t,),
                device_id_type=pl.DeviceIdType.MESH,
            )
        pl.semaphore_wait(barrier, N_DEV - 1)

        sends = []
        for d in (1, 2, 3):
            t = lax.rem(my + d, N_DEV)
            for h in (0, 1):
                rdma = pltpu.make_async_remote_copy(
                    src_ref=x_ref.at[pl.ds(t * M_BLK + h * MH, MH), :],
                    dst_ref=xg_ref.at[my, pl.ds(h * MH, MH), :],
                    send_sem=send_sems.at[d - 1, h],
                    recv_sem=recv_sems.at[d - 1, h],
                    device_id=(t,),
                    device_id_type=pl.DeviceIdType.MESH,
                )
                rdma.start()
                sends.append(rdma)

        def wait_recv(j, h):
            s = src_of(j)
            sem = _SEM_FOR[j]
            pltpu.make_async_remote_copy(
                src_ref=x_ref.at[pl.ds(s * M_BLK + h * MH, MH), :],
                dst_ref=xg_ref.at[s, pl.ds(h * MH, MH), :],
                send_sem=send_sems.at[sem, h],
                recv_sem=recv_sems.at[sem, h],
                device_id=(s,),
                device_id_type=pl.DeviceIdType.MESH,
            ).wait_recv()

        def wdma_start(s, n, slot):
            pltpu.make_async_copy(
                w_ref.at[pl.ds(s * M_BLK, M_BLK), pl.ds(n * NT, NT)],
                w_buf.at[slot], w_sems.at[slot],
            ).start()

        def wdma_wait(slot):
            pltpu.make_async_copy(
                w_ref.at[pl.ds(0, M_BLK), pl.ds(0, NT)],
                w_buf.at[slot], w_sems.at[slot],
            ).wait()

        x0 = pltpu.make_async_copy(
            x_ref.at[pl.ds(my * M_BLK, M_BLK), :], x_full, x_sem
        )
        x0.start()
        s0 = src_of(0)
        wdma_start(s0, 0, 0)
        wdma_start(s0, 1, 1)
        x0.wait()

        def j0_body(n, _):
            slot = lax.rem(n, 2)
            wdma_wait(slot)
            prod = jnp.dot(
                x_full[:, :], w_buf[slot].astype(jnp.bfloat16),
                preferred_element_type=jnp.float32,
            )
            acc_ref[:, pl.ds(n * NT, NT)] = prod

            @pl.when(n + 2 < N_TILES)
            def _():
                wdma_start(s0, n + 2, slot)

            return 0

        lax.fori_loop(0, N_TILES, j0_body, 0)

        for j in (1, 2, 3):
            s = src_of(j)
            for h in (0, 1):
                wdma_start(s, 0, 0)
                wdma_start(s, 1, 1)
                wait_recv(j, h)
                xh = pltpu.make_async_copy(
                    xg_ref.at[s, pl.ds(h * MH, MH), :], x_half, x_sem
                )
                xh.start()
                xh.wait()
                row = slice(h * MH, (h + 1) * MH)

                def jh_body(n, _, j=j, s=s, row=row):
                    slot = lax.rem(n, 2)
                    wdma_wait(slot)
                    prod = jnp.dot(
                        x_half[:, :], w_buf[slot].astype(jnp.bfloat16),
                        preferred_element_type=jnp.float32,
                    )
                    acc_ref[row, pl.ds(n * NT, NT)] += prod

                    @pl.when(n + 2 < N_TILES)
                    def _():
                        wdma_start(s, n + 2, slot)

                    return 0

                lax.fori_loop(0, N_TILES, jh_body, 0)

                if j == 3:
                    def silu_body(n, _, row=row):
                        v = acc_ref[row, pl.ds(n * NT, NT)]
                        acc_ref[row, pl.ds(n * NT, NT)] = v * (
                            1.0 / (1.0 + jnp.exp(-v))
                        )
                        return 0

                    lax.fori_loop(0, N_TILES, silu_body, 0)
                    pltpu.make_async_copy(
                        acc_ref.at[row, :], out_ref.at[row, :], out_sems.at[h]
                    ).start()

        for h in (0, 1):
            pltpu.make_async_copy(
                acc_ref.at[pl.ds(h * MH, MH), :],
                out_ref.at[pl.ds(h * MH, MH), :],
                out_sems.at[h],
            ).wait()

        for rdma in sends:
            rdma.wait_send()

    y, _ = pl.pallas_call(
        body,
        out_shape=[
            jax.ShapeDtypeStruct((M_BLK, n_out), jnp.float32),
            jax.ShapeDtypeStruct((N_DEV, M_BLK, k_loc), jnp.bfloat16),
        ],
        in_specs=[
            pl.BlockSpec(memory_space=pl.ANY),
            pl.BlockSpec(memory_space=pl.ANY),
        ],
        out_specs=[
            pl.BlockSpec(memory_space=pl.ANY),
            pl.BlockSpec(memory_space=pl.ANY),
        ],
        scratch_shapes=[
            pltpu.VMEM((M_BLK, k_loc), jnp.bfloat16),
            pltpu.VMEM((MH, k_loc), jnp.bfloat16),
            pltpu.VMEM((2, M_BLK, NT), jnp.float32),
            pltpu.VMEM((M_BLK, n_out), jnp.float32),
            pltpu.SemaphoreType.DMA((N_DEV - 1, 2)),
            pltpu.SemaphoreType.DMA((N_DEV - 1, 2)),
            pltpu.SemaphoreType.DMA,
            pltpu.SemaphoreType.DMA((2,)),
            pltpu.SemaphoreType.DMA((2,)),
        ],
        compiler_params=pltpu.CompilerParams(
            collective_id=0,
            vmem_limit_bytes=63 * 1024 * 1024,
        ),
    )(x, w_mat)
    return y


# device time: 322319 ns/iter; 2.1978x vs baseline; 1.0529x over previous
import jax
import jax.numpy as jnp
from jax import lax
from jax.experimental import pallas as pl
from jax.experimental.pallas import tpu as pltpu

jax.config.update("jax_compilation_cache_dir", "/tmp/jax_cache")
jax.config.update("jax_persistent_cache_min_compile_time_secs", 0.0)

N_DEV = 4
M_BLK = 2048
MH = 1024
MQ = 512
NT = 512
N_TILES = 8

_SRC_OFF = (0, 3, 1, 2)
_SEM_FOR = (None, 0, 2, 1)


def kernel(x, w_mat):
    k_tot, k_loc = x.shape
    n_out = w_mat.shape[1]
    assert k_loc == M_BLK and n_out == N_TILES * NT

    def body(x_ref, w_ref, out_ref, xg_ref, stage_ref,
             cf32, cbf, xh_f32, x_half, w_buf, acc_ref,
             send_sems, recv_sems, cin_sem, cout_sem, x_sem, w_sems,
             out_sems):
        my = lax.axis_index("i")

        def src_of(j):
            return lax.rem(my + _SRC_OFF[j], N_DEV)

        barrier = pltpu.get_barrier_semaphore()
        for d in (1, 2, 3):
            t = lax.rem(my + d, N_DEV)
            pl.semaphore_signal(
                barrier, inc=1, device_id=(t,),
                device_id_type=pl.DeviceIdType.MESH,
            )
        pl.semaphore_wait(barrier, N_DEV - 1)

        sends = []
        for d in (1, 2, 3):
            t = lax.rem(my + d, N_DEV)
            for h in (0, 1):
                for qh in (0, 1):
                    q = 2 * h + qh
                    cin = pltpu.make_async_copy(
                        x_ref.at[pl.ds(t * M_BLK + q * MQ, MQ), :],
                        cf32, cin_sem,
                    )
                    cin.start()
                    cin.wait()
                    cbf[:, :] = cf32[:, :].astype(jnp.bfloat16)
                    cout = pltpu.make_async_copy(
                        cbf, stage_ref.at[d - 1, pl.ds(q * MQ, MQ), :],
                        cout_sem,
                    )
                    cout.start()
                    cout.wait()
                rdma = pltpu.make_async_remote_copy(
                    src_ref=stage_ref.at[d - 1, pl.ds(h * MH, MH), :],
                    dst_ref=xg_ref.at[my, pl.ds(h * MH, MH), :],
                    send_sem=send_sems.at[d - 1, h],
                    recv_sem=recv_sems.at[d - 1, h],
                    device_id=(t,),
                    device_id_type=pl.DeviceIdType.MESH,
                )
                rdma.start()
                sends.append(rdma)

        def wait_recv(j, h):
            s = src_of(j)
            sem = _SEM_FOR[j]
            pltpu.make_async_remote_copy(
                src_ref=stage_ref.at[0, pl.ds(h * MH, MH), :],
                dst_ref=xg_ref.at[s, pl.ds(h * MH, MH), :],
                send_sem=send_sems.at[sem, h],
                recv_sem=recv_sems.at[sem, h],
                device_id=(s,),
                device_id_type=pl.DeviceIdType.MESH,
            ).wait_recv()

        def wdma_start(s, n, slot):
            pltpu.make_async_copy(
                w_ref.at[pl.ds(s * M_BLK, M_BLK), pl.ds(n * NT, NT)],
                w_buf.at[slot], w_sems.at[slot],
            ).start()

        def wdma_wait(slot):
            pltpu.make_async_copy(
                w_ref.at[pl.ds(0, M_BLK), pl.ds(0, NT)],
                w_buf.at[slot], w_sems.at[slot],
            ).wait()

        s0 = src_of(0)
        for h in (0, 1):
            xh = pltpu.make_async_copy(
                x_ref.at[pl.ds(my * M_BLK + h * MH, MH), :], xh_f32, x_sem
            )
            xh.start()
            wdma_start(s0, 0, 0)
            wdma_start(s0, 1, 1)
            xh.wait()
            row = slice(h * MH, (h + 1) * MH)

            def j0_body(n, _, row=row):
                slot = lax.rem(n, 2)
                wdma_wait(slot)
                prod = jnp.dot(
                    xh_f32[:, :], w_buf[slot],
                    preferred_element_type=jnp.float32,
                )
                acc_ref[row, pl.ds(n * NT, NT)] = prod

                @pl.when(n + 2 < N_TILES)
                def _():
                    wdma_start(s0, n + 2, slot)

                return 0

            lax.fori_loop(0, N_TILES, j0_body, 0)

        for j in (1, 2, 3):
            s = src_of(j)
            for h in (0, 1):
                wdma_start(s, 0, 0)
                wdma_start(s, 1, 1)
                wait_recv(j, h)
                xh = pltpu.make_async_copy(
                    xg_ref.at[s, pl.ds(h * MH, MH), :], x_half, x_sem
                )
                xh.start()
                xh.wait()
                row = slice(h * MH, (h + 1) * MH)

                def jh_body(n, _, j=j, s=s, row=row):
                    slot = lax.rem(n, 2)
                    wdma_wait(slot)
                    prod = jnp.dot(
                        x_half[:, :], w_buf[slot].astype(jnp.bfloat16),
                        preferred_element_type=jnp.float32,
                    )
                    acc_ref[row, pl.ds(n * NT, NT)] += prod

                    @pl.when(n + 2 < N_TILES)
                    def _():
                        wdma_start(s, n + 2, slot)

                    return 0

                lax.fori_loop(0, N_TILES, jh_body, 0)

                if j == 3:
                    def silu_body(n, _, row=row):
                        v = acc_ref[row, pl.ds(n * NT, NT)]
                        acc_ref[row, pl.ds(n * NT, NT)] = v * (
                            1.0 / (1.0 + jnp.exp(-v))
                        )
                        return 0

                    lax.fori_loop(0, N_TILES, silu_body, 0)
                    pltpu.make_async_copy(
                        acc_ref.at[row, :], out_ref.at[row, :], out_sems.at[h]
                    ).start()

        for h in (0, 1):
            pltpu.make_async_copy(
                acc_ref.at[pl.ds(h * MH, MH), :],
                out_ref.at[pl.ds(h * MH, MH), :],
                out_sems.at[h],
            ).wait()

        for rdma in sends:
            rdma.wait_send()

    y, _, _ = pl.pallas_call(
        body,
        out_shape=[
            jax.ShapeDtypeStruct((M_BLK, n_out), jnp.float32),
            jax.ShapeDtypeStruct((N_DEV, M_BLK, k_loc), jnp.bfloat16),
            jax.ShapeDtypeStruct((N_DEV - 1, M_BLK, k_loc), jnp.bfloat16),
        ],
        in_specs=[
            pl.BlockSpec(memory_space=pl.ANY),
            pl.BlockSpec(memory_space=pl.ANY),
        ],
        out_specs=[
            pl.BlockSpec(memory_space=pl.ANY),
            pl.BlockSpec(memory_space=pl.ANY),
            pl.BlockSpec(memory_space=pl.ANY),
        ],
        scratch_shapes=[
            pltpu.VMEM((MQ, k_loc), jnp.float32),
            pltpu.VMEM((MQ, k_loc), jnp.bfloat16),
            pltpu.VMEM((MH, k_loc), jnp.float32),
            pltpu.VMEM((MH, k_loc), jnp.bfloat16),
            pltpu.VMEM((2, M_BLK, NT), jnp.float32),
            pltpu.VMEM((M_BLK, n_out), jnp.float32),
            pltpu.SemaphoreType.DMA((N_DEV - 1, 2)),
            pltpu.SemaphoreType.DMA((N_DEV - 1, 2)),
            pltpu.SemaphoreType.DMA,
            pltpu.SemaphoreType.DMA,
            pltpu.SemaphoreType.DMA,
            pltpu.SemaphoreType.DMA((2,)),
            pltpu.SemaphoreType.DMA((2,)),
        ],
        compiler_params=pltpu.CompilerParams(
            collective_id=0,
            vmem_limit_bytes=63 * 1024 * 1024,
        ),
    )(x, w_mat)
    return y
